# baseline (speedup 1.0000x reference)
import jax
import jax.numpy as jnp
from jax.experimental import pallas as pl
from jax.experimental.pallas import tpu as pltpu

_BN = 768
_D = 64
_DA = 68


def _aug_kernel(x_ref, o_ref, *, is_encoding):
    x = x_ref[...]
    n2 = jnp.sum(x * x, axis=1, keepdims=True)
    hi = n2.astype(jnp.bfloat16)
    lo = (n2 - hi.astype(jnp.float32)).astype(jnp.bfloat16)
    one = jnp.ones_like(hi)
    if is_encoding:
        cols = [(-2.0 * x).astype(jnp.bfloat16), hi, lo, one, one]
        o_ref[...] = jnp.concatenate(cols, axis=1)
    else:
        cols = [x.astype(jnp.bfloat16), one, one, hi, lo]
        o_ref[...] = jnp.concatenate(cols, axis=1).T


def _augment(x, is_encoding, bm):
    m = x.shape[0]
    bm = min(bm, m)
    if is_encoding:
        out_spec = pl.BlockSpec((bm, _DA), lambda i: (i, 0))
        out_shape = jax.ShapeDtypeStruct((m, _DA), jnp.bfloat16)
    else:
        out_spec = pl.BlockSpec((_DA, bm), lambda i: (0, i))
        out_shape = jax.ShapeDtypeStruct((_DA, m), jnp.bfloat16)
    return pl.pallas_call(
        lambda x_ref, o_ref: _aug_kernel(x_ref, o_ref, is_encoding=is_encoding),
        grid=(m // bm,),
        in_specs=[pl.BlockSpec((bm, _D), lambda i: (i, 0))],
        out_specs=out_spec,
        out_shape=out_shape,
    )(x)


def _dist_kernel(e_ref, ca_ref, o_ref):
    x = e_ref[...]
    n2 = jnp.sum(x * x, axis=1, keepdims=True)
    hi = n2.astype(jnp.bfloat16)
    lo = (n2 - hi.astype(jnp.float32)).astype(jnp.bfloat16)
    one = jnp.ones_like(hi)
    ea = jnp.concatenate(
        [(-2.0 * x).astype(jnp.bfloat16), hi, lo, one, one], axis=1)
    o_ref[...] = jax.lax.dot_general(
        ea, ca_ref[...], (((1,), (0,)), ((), ())),
        preferred_element_type=jnp.float32,
    )


def kernel(encoding, codebook):
    n, _ = encoding.shape
    k, _ = codebook.shape
    ca = _augment(codebook, False, 4096)
    grid = (n // _BN,)
    return pl.pallas_call(
        _dist_kernel,
        grid=grid,
        in_specs=[
            pl.BlockSpec((_BN, _D), lambda i: (i, 0)),
            pl.BlockSpec((_DA, k), lambda i: (0, 0)),
        ],
        out_specs=pl.BlockSpec((_BN, k), lambda i: (i, 0)),
        out_shape=jax.ShapeDtypeStruct((n, k), jnp.float32),
        compiler_params=pltpu.CompilerParams(
            dimension_semantics=("parallel",),
            vmem_limit_bytes=63 * 1024 * 1024,
        ),
    )(encoding, ca)


# speedup vs baseline: 1.0177x; 1.0177x over previous
import jax
import jax.numpy as jnp
from jax.experimental import pallas as pl
from jax.experimental.pallas import tpu as pltpu

_BN = 512
_D = 64
_DA = 68


def _hi_lo(n2):
    hi = n2.astype(jnp.bfloat16)
    lo = (n2 - hi.astype(jnp.float32)).astype(jnp.bfloat16)
    return hi, lo


def _dist_kernel(e_ref, cb_ref, o_ref, ca_ref):
    @pl.when(pl.program_id(0) == 0)
    def _():
        c = cb_ref[...]
        c2 = jnp.sum(c * c, axis=1, keepdims=True)
        hi, lo = _hi_lo(c2)
        one = jnp.ones_like(hi)
        ca_ref[...] = jnp.concatenate(
            [c.astype(jnp.bfloat16), one, one, hi, lo], axis=1).T

    x = e_ref[...]
    z2 = jnp.sum(x * x, axis=1, keepdims=True)
    hi, lo = _hi_lo(z2)
    one = jnp.ones_like(hi)
    ea = jnp.concatenate(
        [(-2.0 * x).astype(jnp.bfloat16), hi, lo, one, one], axis=1)
    o_ref[...] = jax.lax.dot_general(
        ea, ca_ref[...], (((1,), (0,)), ((), ())),
        preferred_element_type=jnp.float32,
    )


def kernel(encoding, codebook):
    n, d = encoding.shape
    k, _ = codebook.shape
    return pl.pallas_call(
        _dist_kernel,
        grid=(n // _BN,),
        in_specs=[
            pl.BlockSpec((_BN, d), lambda i: (i, 0)),
            pl.BlockSpec((k, d), lambda i: (0, 0)),
        ],
        out_specs=pl.BlockSpec((_BN, k), lambda i: (i, 0)),
        out_shape=jax.ShapeDtypeStruct((n, k), jnp.float32),
        scratch_shapes=[pltpu.VMEM((_DA, k), jnp.bfloat16)],
        compiler_params=pltpu.CompilerParams(
            dimension_semantics=("arbitrary",),
            vmem_limit_bytes=63 * 1024 * 1024,
        ),
    )(encoding, codebook)
